# baseline (speedup 1.0000x reference)
import functools

import jax
import jax.numpy as jnp
from jax import lax
from jax.experimental import pallas as pl
from jax.experimental.pallas import tpu as pltpu
from jax.experimental.pallas import tpu_sc as plsc

_LANES = 16


@functools.lru_cache(maxsize=None)
def _build_sc_kernel(S, D, T, NC, NS, JC):
    NW = NC * NS
    ROWS = S // NW
    NJC = S // JC
    VPR = D // _LANES

    mesh = plsc.VectorSubcoreMesh(core_axis_name="c", subcore_axis_name="s")

    Q = 8
    NGJ = ROWS // Q
    NG = NJC * NGJ
    TR = JC + Q - 1

    @functools.partial(
        pl.kernel,
        out_type=jax.ShapeDtypeStruct((S, S, D), jnp.float32),
        mesh=mesh,
        scratch_types=[
            pltpu.VMEM((JC, D), jnp.float32),
            [pltpu.VMEM((TR * D,), jnp.float32) for _ in range(2)],
            [pltpu.VMEM((Q, JC, D), jnp.float32) for _ in range(2)],
            [pltpu.SemaphoreType.DMA for _ in range(2)],
            [pltpu.SemaphoreType.DMA for _ in range(2)],
            pltpu.VMEM_SHARED((T * D,), jnp.float32),
            pltpu.VMEM_SHARED((S, D), jnp.float32),
        ],
    )
    def sc_kernel(x_hbm, tab_hbm, out_hbm, xbuf, tbs, obs, tsems, osems,
                  stab, sx):
        sid = lax.axis_index("s")
        wid = sid * NC + lax.axis_index("c")
        i0 = wid * ROWS
        ML = (T - 1) // 2

        @pl.when(sid == 0)
        def _():
            pltpu.sync_copy(tab_hbm, stab)

        @pl.when(sid == 1)
        def _():
            pltpu.sync_copy(x_hbm, sx)

        plsc.subcore_barrier()

        def tstart(r0, j0, b):
            start = r0 + (ML - JC + 1) - j0
            pltpu.make_async_copy(
                stab.at[pl.ds(start * D, TR * D)], tbs[b], tsems[b]).start()

        def twait(b):
            pltpu.make_async_copy(
                stab.at[pl.ds(0, TR * D)], tbs[b], tsems[b]).wait()

        def ostart(r0, j0, b):
            pltpu.make_async_copy(
                obs[b], out_hbm.at[pl.ds(r0, Q), pl.ds(j0, JC)], osems[b]).start()

        def owait(b):
            pltpu.make_async_copy(
                out_hbm.at[pl.ds(0, Q), pl.ds(0, JC)], obs[b], osems[b]).wait()

        def compute_group(p):
            tb = tbs[p]
            og = obs[p]

            @plsc.parallel_loop(0, JC, unroll=1)
            def _(jj):
                base = (JC - 1 - jj) * D
                for v in range(VPR):
                    sl = pl.ds(v * _LANES, _LANES)
                    xv = xbuf[jj, sl]
                    for q in range(Q):
                        og[q, jj, sl] = xv + tb[pl.ds(base + q * D + v * _LANES, _LANES)]

        def group_params(g):
            jc = g // NGJ
            t = g - jc * NGJ
            return i0 + Q * t, jc * JC

        r0p, j0p = group_params(0)
        tstart(r0p, j0p, 0)

        def gloop(g2, _):
            for p in range(2):
                g = 2 * g2 + p
                r0, j0 = group_params(g)

                @pl.when(g % NGJ == 0)
                def _():
                    pltpu.sync_copy(sx.at[pl.ds(j0, JC)], xbuf)

                @pl.when(g < NG - 1)
                def _():
                    nr0, nj0 = group_params(g + 1)
                    tstart(nr0, nj0, 1 - p)

                twait(p)

                @pl.when(g >= 2)
                def _():
                    owait(p)

                compute_group(p)
                ostart(r0, j0, p)
            return 0

        lax.fori_loop(0, NG // 2, gloop, 0)
        for b in range(2):
            owait(b)

    return sc_kernel


def kernel(x, rel_pos_embeddings):
    batch, S, D = x.shape
    T = rel_pos_embeddings.shape[0]
    info = plsc.get_sparse_core_info()
    sc = _build_sc_kernel(S, D, T, info.num_cores, info.num_subcores, 32)
    return sc(x.reshape(S, D), rel_pos_embeddings.reshape(T * D))


# speedup vs baseline: 1.2696x; 1.2696x over previous
import functools

import jax
import jax.numpy as jnp
from jax import lax
from jax.experimental import pallas as pl
from jax.experimental.pallas import tpu as pltpu
from jax.experimental.pallas import tpu_sc as plsc

_LANES = 16


@functools.lru_cache(maxsize=None)
def _build_sc_kernel(S, D, T, NC, NS, JC):
    NW = NC * NS
    ROWS = S // NW
    NJC = S // JC
    VPR = D // _LANES

    mesh = plsc.VectorSubcoreMesh(core_axis_name="c", subcore_axis_name="s")

    Q = 4
    NGJ = ROWS // Q
    NG = NJC * NGJ
    TR = JC + Q - 1

    @functools.partial(
        pl.kernel,
        out_type=jax.ShapeDtypeStruct((S, S, D), jnp.float32),
        mesh=mesh,
        scratch_types=[
            pltpu.VMEM((JC, D), jnp.float32),
            [pltpu.VMEM((TR * D,), jnp.float32) for _ in range(2)],
            [pltpu.VMEM((Q, JC, D), jnp.float32) for _ in range(2)],
            [pltpu.SemaphoreType.DMA for _ in range(2)],
            [pltpu.SemaphoreType.DMA for _ in range(2)],
            pltpu.VMEM_SHARED((T * D,), jnp.float32),
            pltpu.VMEM_SHARED((S, D), jnp.float32),
        ],
    )
    def sc_kernel(x_hbm, tab_hbm, out_hbm, xbuf, tbs, obs, tsems, osems,
                  stab, sx):
        sid = lax.axis_index("s")
        wid = sid * NC + lax.axis_index("c")
        i0 = wid * ROWS
        ML = (T - 1) // 2

        @pl.when(sid == 0)
        def _():
            pltpu.sync_copy(tab_hbm, stab)

        @pl.when(sid == 1)
        def _():
            pltpu.sync_copy(x_hbm, sx)

        plsc.subcore_barrier()

        def tstart(r0, j0, b):
            start = r0 + (ML - JC + 1) - j0
            pltpu.make_async_copy(
                stab.at[pl.ds(start * D, TR * D)], tbs[b], tsems[b]).start()

        def twait(b):
            pltpu.make_async_copy(
                stab.at[pl.ds(0, TR * D)], tbs[b], tsems[b]).wait()

        def ostart(r0, j0, b):
            pltpu.make_async_copy(
                obs[b], out_hbm.at[pl.ds(r0, Q), pl.ds(j0, JC)], osems[b]).start()

        def owait(b):
            pltpu.make_async_copy(
                out_hbm.at[pl.ds(0, Q), pl.ds(0, JC)], obs[b], osems[b]).wait()

        def compute_group(p):
            tb = tbs[p]
            og = obs[p]

            @plsc.parallel_loop(0, JC, unroll=1)
            def _(jj):
                base = (JC - 1 - jj) * D
                for v in range(VPR):
                    sl = pl.ds(v * _LANES, _LANES)
                    xv = xbuf[jj, sl]
                    for q in range(Q):
                        og[q, jj, sl] = xv + tb[pl.ds(base + q * D + v * _LANES, _LANES)]

        def group_params(g):
            jc = g // NGJ
            t = g - jc * NGJ
            return i0 + Q * t, jc * JC

        r0p, j0p = group_params(0)
        tstart(r0p, j0p, 0)

        def gloop(g2, _):
            for p in range(2):
                g = 2 * g2 + p
                r0, j0 = group_params(g)

                @pl.when(g % NGJ == 0)
                def _():
                    pltpu.sync_copy(sx.at[pl.ds(j0, JC)], xbuf)

                @pl.when(g < NG - 1)
                def _():
                    nr0, nj0 = group_params(g + 1)
                    tstart(nr0, nj0, 1 - p)

                twait(p)

                @pl.when(g >= 2)
                def _():
                    owait(p)

                compute_group(p)
                ostart(r0, j0, p)
            return 0

        lax.fori_loop(0, NG // 2, gloop, 0)
        for b in range(2):
            owait(b)

    return sc_kernel


def kernel(x, rel_pos_embeddings):
    batch, S, D = x.shape
    T = rel_pos_embeddings.shape[0]
    info = plsc.get_sparse_core_info()
    sc = _build_sc_kernel(S, D, T, info.num_cores, info.num_subcores, 64)
    return sc(x.reshape(S, D), rel_pos_embeddings.reshape(T * D))


# speedup vs baseline: 1.3314x; 1.0487x over previous
import functools

import jax
import jax.numpy as jnp
from jax import lax
from jax.experimental import pallas as pl
from jax.experimental.pallas import tpu as pltpu
from jax.experimental.pallas import tpu_sc as plsc

_LANES = 16


@functools.lru_cache(maxsize=None)
def _build_sc_kernel(S, D, T, NC, NS, JC):
    NW = NC * NS
    ROWS = S // NW
    NJC = S // JC
    VPR = D // _LANES

    mesh = plsc.VectorSubcoreMesh(core_axis_name="c", subcore_axis_name="s")

    Q = 4
    NGJ = ROWS // Q
    NG = NJC * NGJ
    TR = JC + Q - 1

    @functools.partial(
        pl.kernel,
        out_type=jax.ShapeDtypeStruct((S, S, D), jnp.float32),
        mesh=mesh,
        scratch_types=[
            pltpu.VMEM((JC, D), jnp.float32),
            [pltpu.VMEM((TR * D,), jnp.float32) for _ in range(2)],
            [pltpu.VMEM((Q, JC, D), jnp.float32) for _ in range(2)],
            [pltpu.SemaphoreType.DMA for _ in range(2)],
            [pltpu.SemaphoreType.DMA for _ in range(2)],
            pltpu.VMEM_SHARED((T * D,), jnp.float32),
            pltpu.VMEM_SHARED((S, D), jnp.float32),
        ],
    )
    def sc_kernel(x_hbm, tab_hbm, out_hbm, xbuf, tbs, obs, tsems, osems,
                  stab, sx):
        sid = lax.axis_index("s")
        wid = sid * NC + lax.axis_index("c")
        i0 = wid * ROWS
        ML = (T - 1) // 2

        @pl.when(sid == 0)
        def _():
            pltpu.sync_copy(tab_hbm, stab)

        @pl.when(sid == 1)
        def _():
            pltpu.sync_copy(x_hbm, sx)

        plsc.subcore_barrier()

        def tstart(r0, j0, b):
            start = r0 + (ML - JC + 1) - j0
            pltpu.make_async_copy(
                stab.at[pl.ds(start * D, TR * D)], tbs[b], tsems[b]).start()

        def twait(b):
            pltpu.make_async_copy(
                stab.at[pl.ds(0, TR * D)], tbs[b], tsems[b]).wait()

        def ostart(r0, j0, b):
            pltpu.make_async_copy(
                obs[b], out_hbm.at[pl.ds(r0, Q), pl.ds(j0, JC)], osems[b]).start()

        def owait(b):
            pltpu.make_async_copy(
                out_hbm.at[pl.ds(0, Q), pl.ds(0, JC)], obs[b], osems[b]).wait()

        def compute_group(p):
            tb = tbs[p]
            og = obs[p]

            def trow(k):
                return tuple(
                    tb[pl.ds(k * D + v * _LANES, _LANES)] for v in range(VPR))

            init = tuple(trow(JC - 1 + q) for q in range(Q))

            @plsc.parallel_loop(0, JC, carry=init)
            def _(jj, tcar):
                for v in range(VPR):
                    sl = pl.ds(v * _LANES, _LANES)
                    xv = xbuf[jj, sl]
                    for q in range(Q):
                        og[q, jj, sl] = xv + tcar[q][v]
                nk = jnp.maximum(JC - 2 - jj, 0)
                return (trow(nk),) + tcar[:Q - 1]

        def group_params(g):
            jc = g // NGJ
            t = g - jc * NGJ
            return i0 + Q * t, jc * JC

        r0p, j0p = group_params(0)
        tstart(r0p, j0p, 0)

        def gloop(g2, _):
            for p in range(2):
                g = 2 * g2 + p
                r0, j0 = group_params(g)

                @pl.when(g % NGJ == 0)
                def _():
                    pltpu.sync_copy(sx.at[pl.ds(j0, JC)], xbuf)

                @pl.when(g < NG - 1)
                def _():
                    nr0, nj0 = group_params(g + 1)
                    tstart(nr0, nj0, 1 - p)

                twait(p)

                @pl.when(g >= 2)
                def _():
                    owait(p)

                compute_group(p)
                ostart(r0, j0, p)
            return 0

        lax.fori_loop(0, NG // 2, gloop, 0)
        for b in range(2):
            owait(b)

    return sc_kernel


def kernel(x, rel_pos_embeddings):
    batch, S, D = x.shape
    T = rel_pos_embeddings.shape[0]
    info = plsc.get_sparse_core_info()
    sc = _build_sc_kernel(S, D, T, info.num_cores, info.num_subcores, 64)
    return sc(x.reshape(S, D), rel_pos_embeddings.reshape(T * D))


# speedup vs baseline: 1.3318x; 1.0003x over previous
import functools

import jax
import jax.numpy as jnp
from jax import lax
from jax.experimental import pallas as pl
from jax.experimental.pallas import tpu as pltpu
from jax.experimental.pallas import tpu_sc as plsc

_LANES = 16


@functools.lru_cache(maxsize=None)
def _build_sc_kernel(S, D, T, NC, NS, JC):
    NW = NC * NS
    ROWS = S // NW
    NJC = S // JC
    VPR = D // _LANES

    mesh = plsc.VectorSubcoreMesh(core_axis_name="c", subcore_axis_name="s")

    Q = 4
    NGJ = ROWS // Q
    NG = NJC * NGJ
    TR = JC + Q - 1

    @functools.partial(
        pl.kernel,
        out_type=jax.ShapeDtypeStruct((S, S, D), jnp.float32),
        mesh=mesh,
        scratch_types=[
            pltpu.VMEM((JC, D), jnp.float32),
            [pltpu.VMEM((TR * D,), jnp.float32) for _ in range(2)],
            [pltpu.VMEM((Q, JC, D), jnp.float32) for _ in range(2)],
            [pltpu.SemaphoreType.DMA for _ in range(2)],
            [pltpu.SemaphoreType.DMA for _ in range(2)],
            pltpu.VMEM_SHARED((T * D,), jnp.float32),
            pltpu.VMEM_SHARED((S, D), jnp.float32),
        ],
    )
    def sc_kernel(x_hbm, tab_hbm, out_hbm, xbuf, tbs, obs, tsems, osems,
                  stab, sx):
        sid = lax.axis_index("s")
        wid = sid * NC + lax.axis_index("c")
        i0 = wid * ROWS
        ML = (T - 1) // 2

        @pl.when(sid == 0)
        def _():
            pltpu.sync_copy(tab_hbm, stab)

        @pl.when(sid == 1)
        def _():
            pltpu.sync_copy(x_hbm, sx)

        plsc.subcore_barrier()

        def tstart(r0, j0, b):
            start = r0 + (ML - JC + 1) - j0
            pltpu.make_async_copy(
                stab.at[pl.ds(start * D, TR * D)], tbs[b], tsems[b]).start()

        def twait(b):
            pltpu.make_async_copy(
                stab.at[pl.ds(0, TR * D)], tbs[b], tsems[b]).wait()

        def ostart(r0, j0, b):
            pltpu.make_async_copy(
                obs[b], out_hbm.at[pl.ds(r0, Q), pl.ds(j0, JC)], osems[b]).start()

        def owait(b):
            pltpu.make_async_copy(
                out_hbm.at[pl.ds(0, Q), pl.ds(0, JC)], obs[b], osems[b]).wait()

        def compute_group(p):
            tb = tbs[p]
            og = obs[p]

            def trow(k):
                return tuple(
                    tb[pl.ds(k * D + v * _LANES, _LANES)] for v in range(VPR))

            init = tuple(trow(JC - 1 + q) for q in range(Q))

            @plsc.parallel_loop(0, JC, unroll=2, carry=init)
            def _(jj, tcar):
                for v in range(VPR):
                    sl = pl.ds(v * _LANES, _LANES)
                    xv = xbuf[jj, sl]
                    for q in range(Q):
                        og[q, jj, sl] = xv + tcar[q][v]
                nk = jnp.maximum(JC - 2 - jj, 0)
                return (trow(nk),) + tcar[:Q - 1]

        def group_params(g):
            jc = g // NGJ
            t = g - jc * NGJ
            return i0 + Q * t, jc * JC

        r0p, j0p = group_params(0)
        tstart(r0p, j0p, 0)

        def gloop(g2, _):
            for p in range(2):
                g = 2 * g2 + p
                r0, j0 = group_params(g)

                @pl.when(g % NGJ == 0)
                def _():
                    pltpu.sync_copy(sx.at[pl.ds(j0, JC)], xbuf)

                @pl.when(g < NG - 1)
                def _():
                    nr0, nj0 = group_params(g + 1)
                    tstart(nr0, nj0, 1 - p)

                twait(p)

                @pl.when(g >= 2)
                def _():
                    owait(p)

                compute_group(p)
                ostart(r0, j0, p)
            return 0

        lax.fori_loop(0, NG // 2, gloop, 0)
        for b in range(2):
            owait(b)

    return sc_kernel


def kernel(x, rel_pos_embeddings):
    batch, S, D = x.shape
    T = rel_pos_embeddings.shape[0]
    info = plsc.get_sparse_core_info()
    sc = _build_sc_kernel(S, D, T, info.num_cores, info.num_subcores, 64)
    return sc(x.reshape(S, D), rel_pos_embeddings.reshape(T * D))


# speedup vs baseline: 1.3338x; 1.0015x over previous
import functools

import jax
import jax.numpy as jnp
from jax import lax
from jax.experimental import pallas as pl
from jax.experimental.pallas import tpu as pltpu
from jax.experimental.pallas import tpu_sc as plsc

_LANES = 16


@functools.lru_cache(maxsize=None)
def _build_sc_kernel(S, D, T, NC, NS, JC):
    NW = NC * NS
    ROWS = S // NW
    NJC = S // JC
    VPR = D // _LANES

    mesh = plsc.VectorSubcoreMesh(core_axis_name="c", subcore_axis_name="s")

    Q = 4
    NGJ = ROWS // Q
    NG = NJC * NGJ
    TR = JC + Q - 1

    @functools.partial(
        pl.kernel,
        out_type=jax.ShapeDtypeStruct((S, S, D), jnp.float32),
        mesh=mesh,
        scratch_types=[
            pltpu.VMEM((JC, D), jnp.float32),
            [pltpu.VMEM((TR * D,), jnp.float32) for _ in range(2)],
            [pltpu.VMEM((Q, JC, D), jnp.float32) for _ in range(2)],
            [pltpu.SemaphoreType.DMA for _ in range(2)],
            [pltpu.SemaphoreType.DMA for _ in range(2)],
            pltpu.VMEM_SHARED((T * D,), jnp.float32),
            pltpu.VMEM_SHARED((S, D), jnp.float32),
        ],
    )
    def sc_kernel(x_hbm, tab_hbm, out_hbm, xbuf, tbs, obs, tsems, osems,
                  stab, sx):
        sid = lax.axis_index("s")
        wid = sid * NC + lax.axis_index("c")
        i0 = wid * ROWS
        ML = (T - 1) // 2

        @pl.when(sid == 0)
        def _():
            pltpu.sync_copy(tab_hbm, stab)

        @pl.when(sid == 1)
        def _():
            pltpu.sync_copy(x_hbm, sx)

        plsc.subcore_barrier()

        def tstart(r0, j0, b):
            start = r0 + (ML - JC + 1) - j0
            pltpu.make_async_copy(
                stab.at[pl.ds(start * D, TR * D)], tbs[b], tsems[b]).start()

        def twait(b):
            pltpu.make_async_copy(
                stab.at[pl.ds(0, TR * D)], tbs[b], tsems[b]).wait()

        def ostart(r0, j0, b):
            pltpu.make_async_copy(
                obs[b], out_hbm.at[pl.ds(r0, Q), pl.ds(j0, JC)], osems[b]).start()

        def owait(b):
            pltpu.make_async_copy(
                out_hbm.at[pl.ds(0, Q), pl.ds(0, JC)], obs[b], osems[b]).wait()

        def compute_group(p):
            tb = tbs[p]
            og = obs[p]

            def trow(k):
                return tuple(
                    tb[pl.ds(k * D + v * _LANES, _LANES)] for v in range(VPR))

            init = tuple(trow(JC - 1 + q) for q in range(Q))

            @plsc.parallel_loop(0, JC, carry=init)
            def _(jj, tcar):
                for v in range(VPR):
                    sl = pl.ds(v * _LANES, _LANES)
                    xv = xbuf[jj, sl]
                    for q in range(Q):
                        og[q, jj, sl] = xv + tcar[q][v]
                nk = jnp.maximum(JC - 2 - jj, 0)
                return (trow(nk),) + tcar[:Q - 1]

        def group_params(g):
            jc = g // NGJ
            t = g - jc * NGJ
            return i0 + Q * t, jc * JC

        r0p, j0p = group_params(0)
        tstart(r0p, j0p, 0)

        def gloop(g2, _):
            for p in range(2):
                g = 2 * g2 + p
                r0, j0 = group_params(g)

                @pl.when(g % NGJ == 0)
                def _():
                    pltpu.sync_copy(sx.at[pl.ds(j0, JC)], xbuf)

                @pl.when(g < NG - 1)
                def _():
                    nr0, nj0 = group_params(g + 1)
                    tstart(nr0, nj0, 1 - p)

                twait(p)

                @pl.when(g >= 2)
                def _():
                    owait(p)

                compute_group(p)
                ostart(r0, j0, p)
            return 0

        lax.fori_loop(0, NG // 2, gloop, 0)
        for b in range(2):
            owait(b)

    return sc_kernel


def kernel(x, rel_pos_embeddings):
    batch, S, D = x.shape
    T = rel_pos_embeddings.shape[0]
    info = plsc.get_sparse_core_info()
    sc = _build_sc_kernel(S, D, T, info.num_cores, info.num_subcores, 64)
    return sc(x.reshape(S, D), rel_pos_embeddings.reshape(T * D))
